# baseline (speedup 1.0000x reference)
import functools

import jax
import jax.numpy as jnp
from jax import lax
from jax.experimental import pallas as pl
from jax.experimental.pallas import tpu as pltpu
from jax.experimental.pallas import tpu_sc as plsc

D = 8192
K = 4


@functools.lru_cache(maxsize=None)
def _make_kernel(B):
    info = plsc.get_sparse_core_info()
    nc, ns = info.num_cores, info.num_subcores
    nw = nc * ns
    b_per_w = B // nw
    n_chunks = b_per_w // K
    n_half = n_chunks // 2

    mesh = plsc.VectorSubcoreMesh(core_axis_name="c", subcore_axis_name="s")

    @functools.partial(
        pl.kernel,
        mesh=mesh,
        out_type=jax.ShapeDtypeStruct((B, D), jnp.float32),
        scratch_types=[
            pltpu.VMEM((n_chunks, K), jnp.int32),
            pltpu.VMEM((2, K, D), jnp.float32),
            pltpu.SemaphoreType.DMA,
            pltpu.SemaphoreType.DMA,
            pltpu.SemaphoreType.DMA,
            pltpu.SemaphoreType.DMA,
        ],
    )
    def gather_kernel(idx_hbm, table_hbm, out_hbm, idx_v, buf,
                      gsem0, gsem1, wsem0, wsem1):
        wid = lax.axis_index("s") * nc + lax.axis_index("c")
        base = wid * b_per_w
        gsems = (gsem0, gsem1)
        wsems = (wsem0, wsem1)
        pltpu.sync_copy(idx_hbm.at[wid], idx_v)

        def gather_start(g, b):
            pltpu.async_copy(table_hbm.at[idx_v.at[g]], buf.at[b], gsems[b])

        def gather_wait(g, b):
            pltpu.make_async_copy(
                table_hbm.at[idx_v.at[g]], buf.at[b], gsems[b]).wait()

        def write_start(g, b):
            pltpu.async_copy(
                buf.at[b], out_hbm.at[pl.ds(base + g * K, K)], wsems[b])

        def write_wait(g, b):
            pltpu.make_async_copy(
                buf.at[b], out_hbm.at[pl.ds(base + g * K, K)], wsems[b]).wait()

        gather_start(0, 0)
        gather_start(1, 1)

        def chunk_step(g, b, prefetch):
            gather_wait(g, b)
            write_start(g, b)
            if prefetch:
                write_wait(g, b)
                gather_start(g + 2, b)

        def body(i, carry):
            g = 2 * i
            chunk_step(g, 0, True)
            chunk_step(g + 1, 1, True)
            return carry

        lax.fori_loop(0, n_half - 1, body, 0)
        g_last = 2 * (n_half - 1)
        chunk_step(g_last, 0, False)
        chunk_step(g_last + 1, 1, False)
        write_wait(g_last, 0)
        write_wait(g_last + 1, 1)

    return gather_kernel, nw


def kernel(idx, table):
    b, s = idx.shape
    flat = b * s
    gather_kernel, nw = _make_kernel(flat)
    idx_r = idx.reshape(nw, (flat // nw) // K, K)
    out = gather_kernel(idx_r, table)
    return out.reshape(b, s, D)
